# Initial kernel scaffold; baseline (speedup 1.0000x reference)
import jax
import jax.numpy as jnp
from jax.experimental import pallas as pl


def kernel(x, breakpoints, values):
    raise NotImplementedError("write your pallas kernel here")


# speedup vs baseline: 17.1009x; 17.1009x over previous
import jax
import jax.numpy as jnp
from jax.experimental import pallas as pl
from jax.experimental.pallas import tpu as pltpu

_NB = 128
_BW = 16
_BLK = 4096

_EXP_MASK = -65536


def _kernel(x_ref, bnd_ref, bpr_ref, vmain_ref, vext_ref, o_ref):
    f32 = jnp.float32

    bp = bpr_ref[...]
    bits = jax.lax.bitcast_convert_type(bp, jnp.int32)
    hi = jax.lax.bitcast_convert_type(bits & _EXP_MASK, f32)
    r1 = bp - hi
    b1 = jax.lax.bitcast_convert_type(r1, jnp.int32)
    mid = jax.lax.bitcast_convert_type(b1 & _EXP_MASK, f32)
    lo = r1 - mid
    vmain = vmain_ref[...]
    vext = vext_ref[...]
    delta = vext - vmain
    base = vmain[:, 0:1]
    table = jnp.concatenate([hi, mid, lo, delta, base], axis=1)

    xv = x_ref[...]
    h1 = (xv > bnd_ref[...]).astype(f32)
    hs = jnp.concatenate([jnp.ones((xv.shape[0], 1), f32), h1[:, : _NB - 1]], axis=1)
    onehot = hs - h1

    g = jnp.dot(onehot, table, preferred_element_type=f32)
    bp_row = (g[:, 0:16] + g[:, 16:32]) + g[:, 32:48]
    d_row = g[:, 48:64]
    base_row = g[:, 64:65]

    cmp = (xv > bp_row).astype(f32)
    lvl2 = jnp.sum(d_row * cmp, axis=1, keepdims=True)

    v_last = vext_ref[_NB - 1, _BW - 1]
    o_ref[...] = base_row + lvl2 + h1[:, _NB - 1 : _NB] * v_last


def kernel(x, breakpoints, values):
    B = x.shape[0]
    n = breakpoints.shape[0]
    bp_r = breakpoints.reshape(_NB, _BW)
    bnd = bp_r[:, _BW - 1].reshape(1, _NB)
    v_main = values[:n, 0].reshape(_NB, _BW)
    v_ext = values[1 : n + 1, 0].reshape(_NB, _BW)

    grid = (B // _BLK,)
    out = pl.pallas_call(
        _kernel,
        out_shape=jax.ShapeDtypeStruct((B, 1), jnp.float32),
        grid=grid,
        in_specs=[
            pl.BlockSpec((_BLK, 1), lambda i: (i, 0)),
            pl.BlockSpec((1, _NB), lambda i: (0, 0)),
            pl.BlockSpec((_NB, _BW), lambda i: (0, 0)),
            pl.BlockSpec((_NB, _BW), lambda i: (0, 0)),
            pl.BlockSpec((_NB, _BW), lambda i: (0, 0)),
        ],
        out_specs=pl.BlockSpec((_BLK, 1), lambda i: (i, 0)),
        compiler_params=pltpu.CompilerParams(
            dimension_semantics=("parallel",),
        ),
        name="stepnet_lookup",
    )(x, bnd, bp_r, v_main, v_ext)
    return out
